# baseline (speedup 1.0000x reference)
from functools import partial

import jax
import jax.numpy as jnp
from jax.experimental import pallas as pl
from jax.experimental.pallas import tpu as pltpu


def _fused_kernel(x_ref, pm_ref, m1a_ref, m1b_ref, p1a_ref, p1b_ref,
                  m2_ref, p2_ref, m3_ref, s6_ref, w1s_ref, w2_ref, o_ref):
    f32, bf16 = jnp.float32, jnp.bfloat16
    B = x_ref.shape[0]
    L = 8 * B

    def mm(a, b):
        return jnp.dot(a, b, preferred_element_type=f32)

    def shift(a):
        return jnp.concatenate([a[1:], a[:1]], axis=0)

    def pool(u, v):
        a = jnp.maximum(u, 0.0).astype(bf16).astype(f32)
        b = jnp.maximum(v, 0.0).astype(bf16).astype(f32)
        return (a + b).astype(bf16)

    def conv_operand(seq, nphase):
        chunks = [jnp.concatenate(seq[k:k + 3], axis=1)
                  for k in range(nphase)]
        return jnp.concatenate(chunks, axis=0)

    mc1a = mm(pm_ref[...], m1a_ref[...].reshape(384, 256)).astype(bf16)
    mc1b = mm(pm_ref[...], m1b_ref[...].reshape(384, 240)).astype(bf16)

    ch = [x_ref[:, c].reshape(L, 512) for c in range(2)]
    sh = [shift(t) for t in ch]
    chunks = [jnp.concatenate(
        [ch[0][:, 64 * k:64 * k + 192], ch[1][:, 64 * k:64 * k + 192]],
        axis=1) for k in range(6)]
    chunks.append(jnp.concatenate(
        [ch[0][:, 384:512], sh[0][:, 0:64],
         ch[1][:, 384:512], sh[1][:, 0:64]], axis=1))
    chunks.append(jnp.concatenate(
        [ch[0][:, 448:512], sh[0][:, 0:128],
         ch[1][:, 448:512], sh[1][:, 0:128]], axis=1))
    u1 = jnp.concatenate(chunks, axis=0)
    c1a = mm(u1, mc1a)
    c1b = mm(u1, mc1b)
    a1 = [jnp.concatenate(
        [mm(pool(c1a[(2 * p) * L:(2 * p + 1) * L],
                 c1a[(2 * p + 1) * L:(2 * p + 2) * L]), p1a_ref[...]),
         mm(pool(c1b[(2 * p) * L:(2 * p + 1) * L],
                 c1b[(2 * p + 1) * L:(2 * p + 2) * L]), p1b_ref[...])],
        axis=1).astype(bf16) for p in range(4)]

    seq2 = a1 + [shift(a1[0]), shift(a1[1])]
    c2 = mm(conv_operand(seq2, 4), m2_ref[...].reshape(744, 348))
    p2 = [pool(c2[(2 * p) * L:(2 * p + 1) * L],
               c2[(2 * p + 1) * L:(2 * p + 2) * L]) for p in range(2)]
    a2 = [mm(t, p2_ref[...]).astype(bf16) for t in p2]

    seq3 = a2 + [shift(a2[0]), shift(a2[1])]
    c3 = mm(conv_operand(seq3, 2), m3_ref[...].reshape(504, 144))
    rp = pool(c3[0:L], c3[L:2 * L])

    rg = mm(s6_ref[...], rp)
    hin = jnp.concatenate([rg[ip * B:(ip + 1) * B] for ip in range(6)],
                          axis=1).astype(bf16)
    hb = jnp.maximum(mm(hin, w1s_ref[...]), 0.0).astype(bf16)

    o_ref[...] = jnp.maximum(mm(hb, w2_ref[...]), 0.0)


@partial(jax.jit, static_argnames=("block_b",))
def _forward(x_nchw, m1a, m1b, p1a, p1b, m2, p2, m3, w1s, w2fc, block_b=64):
    n, cin, h, w = x_nchw.shape
    assert (cin, h, w) == (2, 64, 64)

    bb = max(1, min(block_b, n))
    n_pad = -(-n // bb) * bb
    L = 8 * bb

    if n_pad != n:
        x_nchw = jnp.pad(x_nchw, ((0, n_pad - n), (0, 0), (0, 0), (0, 0)))
    x6 = x_nchw.astype(jnp.bfloat16).reshape(n_pad, 2, 8, 512)

    r1i = jnp.arange(384)[:, None]
    c1i = jnp.arange(384)[None, :]
    pm = (c1i == ((r1i // 64) % 3) * 128 + 2 * (r1i % 64) + r1i // 192)
    pm = pm.astype(jnp.bfloat16)

    row = jnp.arange(6 * bb)[:, None]
    col = jnp.arange(L)[None, :]
    s6 = (col == 8 * (row % bb) + row // bb).astype(jnp.bfloat16)

    grid = (n_pad // bb,)
    weights = [pm, m1a, m1b, p1a, p1b, m2, p2, m3, s6, w1s, w2fc]

    def const_spec(arr):
        nd = arr.ndim
        return pl.BlockSpec(arr.shape, lambda i, _nd=nd: (0,) * _nd)

    macs_per_img = (64 * 384 * 496 + 32 * 496 * 248 // 2
                    + 32 * 744 * 348 + 16 * 348 * 168
                    + 16 * 504 * 144 + 6 * 144 + 864 * 32 + 32 * 128)
    bytes_accessed = (x6.size * 2 + n_pad * 128 * 4
                      + sum(int(a.size) * 2 for a in weights))

    out = pl.pallas_call(
        _fused_kernel,
        out_shape=jax.ShapeDtypeStruct((n_pad, 128), jnp.float32),
        grid=grid,
        in_specs=[pl.BlockSpec((bb, 2, 8, 512), lambda i: (i, 0, 0, 0))]
                 + [const_spec(a) for a in weights],
        out_specs=pl.BlockSpec((bb, 128), lambda i: (i, 0)),
        compiler_params=pltpu.CompilerParams(
            dimension_semantics=("parallel",)),
        cost_estimate=pl.CostEstimate(
            flops=2 * macs_per_img * n_pad,
            transcendentals=0,
            bytes_accessed=int(bytes_accessed)),
    )(x6, *weights)

    return out[:n, :1]


def kernel(x, m1a, m1b, r1, p1a, p1b, m2, r2, p2, m3, r3, w1fc, w2fc):
    del r1, r2, r3
    half = jnp.asarray(0.5, jnp.bfloat16)
    return _forward(x, m1a, m1b, p1a * half, p1b * half, m2, p2 * half, m3,
                    (w1fc * half).reshape(864, 32), w2fc, block_b=64)


# speedup vs baseline: 1.0447x; 1.0447x over previous
from functools import partial

import jax
import jax.numpy as jnp
from jax.experimental import pallas as pl
from jax.experimental.pallas import tpu as pltpu


def _fused_kernel(x_ref, pm_ref, m1a_ref, m1b_ref, p1a_ref, p1b_ref,
                  m2_ref, p2_ref, m3_ref, s6_ref, w1s_ref, w2_ref, o_ref):
    f32, bf16 = jnp.float32, jnp.bfloat16
    B = x_ref.shape[0]
    L = 8 * B

    def mm(a, b):
        return jnp.dot(a, b, preferred_element_type=f32)

    def shift(a):
        return jnp.concatenate([a[1:], a[:1]], axis=0)

    def pool(u, v):
        a = jnp.maximum(u, 0.0).astype(bf16).astype(f32)
        b = jnp.maximum(v, 0.0).astype(bf16).astype(f32)
        return (a + b).astype(bf16)

    def conv_operand(seq, nphase):
        chunks = [jnp.concatenate(seq[k:k + 3], axis=1)
                  for k in range(nphase)]
        return jnp.concatenate(chunks, axis=0)

    mc1a = mm(pm_ref[...], m1a_ref[...].reshape(384, 256)).astype(bf16)
    mc1b = mm(pm_ref[...], m1b_ref[...].reshape(384, 240)).astype(bf16)

    ch = [x_ref[:, c].reshape(L, 512).astype(bf16) for c in range(2)]
    sh = [shift(t) for t in ch]
    chunks = [jnp.concatenate(
        [ch[0][:, 64 * k:64 * k + 192], ch[1][:, 64 * k:64 * k + 192]],
        axis=1) for k in range(6)]
    chunks.append(jnp.concatenate(
        [ch[0][:, 384:512], sh[0][:, 0:64],
         ch[1][:, 384:512], sh[1][:, 0:64]], axis=1))
    chunks.append(jnp.concatenate(
        [ch[0][:, 448:512], sh[0][:, 0:128],
         ch[1][:, 448:512], sh[1][:, 0:128]], axis=1))
    u1 = jnp.concatenate(chunks, axis=0)
    c1a = mm(u1, mc1a)
    c1b = mm(u1, mc1b)
    ta = jnp.concatenate([pool(c1a[(2 * p) * L:(2 * p + 1) * L],
                               c1a[(2 * p + 1) * L:(2 * p + 2) * L])
                          for p in range(4)], axis=0)
    tb = jnp.concatenate([pool(c1b[(2 * p) * L:(2 * p + 1) * L],
                               c1b[(2 * p + 1) * L:(2 * p + 2) * L])
                          for p in range(4)], axis=0)
    a1f = jnp.concatenate([mm(ta, p1a_ref[...]), mm(tb, p1b_ref[...])],
                          axis=1).astype(bf16)
    a1 = [a1f[p * L:(p + 1) * L] for p in range(4)]

    seq2 = a1 + [shift(a1[0]), shift(a1[1])]
    c2 = mm(conv_operand(seq2, 4), m2_ref[...].reshape(744, 348))
    p2 = [pool(c2[(2 * p) * L:(2 * p + 1) * L],
               c2[(2 * p + 1) * L:(2 * p + 2) * L]) for p in range(2)]
    a2 = [mm(t, p2_ref[...]).astype(bf16) for t in p2]

    seq3 = a2 + [shift(a2[0]), shift(a2[1])]
    c3 = mm(conv_operand(seq3, 2), m3_ref[...].reshape(504, 144))
    rp = pool(c3[0:L], c3[L:2 * L])

    rg = mm(s6_ref[...], rp)
    hin = jnp.concatenate([rg[ip * B:(ip + 1) * B] for ip in range(6)],
                          axis=1).astype(bf16)
    hb = jnp.maximum(mm(hin, w1s_ref[...]), 0.0).astype(bf16)

    o_ref[...] = jnp.maximum(mm(hb, w2_ref[...]), 0.0)


@partial(jax.jit, static_argnames=("block_b",))
def _forward(x_nchw, m1a, m1b, p1a, p1b, m2, p2, m3, w1s, w2fc, block_b=64):
    n, cin, h, w = x_nchw.shape
    assert (cin, h, w) == (2, 64, 64)

    bb = max(1, min(block_b, n))
    n_pad = -(-n // bb) * bb
    L = 8 * bb

    if n_pad != n:
        x_nchw = jnp.pad(x_nchw, ((0, n_pad - n), (0, 0), (0, 0), (0, 0)))
    x6 = x_nchw.reshape(n_pad, 2, 8, 512)

    r1i = jnp.arange(384)[:, None]
    c1i = jnp.arange(384)[None, :]
    pm = c1i == ((r1i // 64) % 3) * 128 + 2 * (r1i % 64) + r1i // 192
    pm = jnp.where(pm, 0.5, 0.0).astype(jnp.bfloat16)

    row = jnp.arange(6 * bb)[:, None]
    col = jnp.arange(L)[None, :]
    s6 = (col == 8 * (row % bb) + row // bb).astype(jnp.bfloat16)

    grid = (n_pad // bb,)
    weights = [pm, m1a, m1b, p1a, p1b, m2, p2, m3, s6, w1s, w2fc]

    def const_spec(arr):
        nd = arr.ndim
        return pl.BlockSpec(arr.shape, lambda i, _nd=nd: (0,) * _nd)

    macs_per_img = (64 * 384 * 496 + 32 * 496 * 248 // 2
                    + 32 * 744 * 348 + 16 * 348 * 168
                    + 16 * 504 * 144 + 6 * 144 + 864 * 32 + 32 * 128)
    bytes_accessed = (x6.size * 4 + n_pad * 128 * 4
                      + sum(int(a.size) * 2 for a in weights))

    out = pl.pallas_call(
        _fused_kernel,
        out_shape=jax.ShapeDtypeStruct((n_pad, 128), jnp.float32),
        grid=grid,
        in_specs=[pl.BlockSpec((bb, 2, 8, 512), lambda i: (i, 0, 0, 0))]
                 + [const_spec(a) for a in weights],
        out_specs=pl.BlockSpec((bb, 128), lambda i: (i, 0)),
        compiler_params=pltpu.CompilerParams(
            dimension_semantics=("parallel",)),
        cost_estimate=pl.CostEstimate(
            flops=2 * macs_per_img * n_pad,
            transcendentals=0,
            bytes_accessed=int(bytes_accessed)),
    )(x6, *weights)

    return out[:n, :1]


def kernel(x, m1a, m1b, r1, p1a, p1b, m2, r2, p2, m3, r3, w1fc, w2fc):
    del r1, r2, r3
    half = jnp.asarray(0.5, jnp.bfloat16)
    return _forward(x, m1a, m1b, p1a, p1b, m2, p2 * half, m3,
                    (w1fc * half).reshape(864, 32), w2fc, block_b=64)


# speedup vs baseline: 1.0905x; 1.0438x over previous
from functools import partial

import jax
import jax.numpy as jnp
from jax.experimental import pallas as pl
from jax.experimental.pallas import tpu as pltpu


def _fused_kernel(x_ref, pm_ref, m1a_ref, m1b_ref, p1a_ref, p1b_ref,
                  m2_ref, p2_ref, m3_ref, s6_ref, w1s_ref, w2_ref, o_ref):
    f32, bf16 = jnp.float32, jnp.bfloat16
    B = x_ref.shape[0]
    L = 8 * B

    def mm(a, b):
        return jnp.dot(a, b, preferred_element_type=f32)

    def shift(a):
        return jnp.concatenate([a[1:], a[:1]], axis=0)

    def pool(u, v):
        a = jnp.maximum(u, 0.0).astype(bf16).astype(f32)
        b = jnp.maximum(v, 0.0).astype(bf16).astype(f32)
        return (a + b).astype(bf16)

    def conv_operand(seq, nphase):
        chunks = [jnp.concatenate(seq[k:k + 3], axis=1)
                  for k in range(nphase)]
        return jnp.concatenate(chunks, axis=0)

    mc1a = mm(pm_ref[...], m1a_ref[...].reshape(384, 256)).astype(bf16)
    mc1b = mm(pm_ref[...], m1b_ref[...].reshape(384, 240)).astype(bf16)

    ch = [x_ref[:, c].reshape(L, 512).astype(bf16) for c in range(2)]
    sh = [shift(t) for t in ch]
    chunks = [jnp.concatenate(
        [ch[0][:, 64 * k:64 * k + 192], ch[1][:, 64 * k:64 * k + 192]],
        axis=1) for k in range(6)]
    chunks.append(jnp.concatenate(
        [ch[0][:, 384:512], sh[0][:, 0:64],
         ch[1][:, 384:512], sh[1][:, 0:64]], axis=1))
    chunks.append(jnp.concatenate(
        [ch[0][:, 448:512], sh[0][:, 0:128],
         ch[1][:, 448:512], sh[1][:, 0:128]], axis=1))
    u1 = jnp.concatenate(chunks, axis=0)
    c1a = mm(u1, mc1a)
    c1b = mm(u1, mc1b)
    ta = jnp.concatenate([pool(c1a[(2 * p) * L:(2 * p + 1) * L],
                               c1a[(2 * p + 1) * L:(2 * p + 2) * L])
                          for p in range(4)], axis=0)
    tb = jnp.concatenate([pool(c1b[(2 * p) * L:(2 * p + 1) * L],
                               c1b[(2 * p + 1) * L:(2 * p + 2) * L])
                          for p in range(4)], axis=0)
    a1f = jnp.concatenate([mm(ta, p1a_ref[...]), mm(tb, p1b_ref[...])],
                          axis=1).astype(bf16)
    a1 = [a1f[p * L:(p + 1) * L] for p in range(4)]

    seq2 = a1 + [shift(a1[0]), shift(a1[1])]
    c2 = mm(conv_operand(seq2, 4), m2_ref[...].reshape(744, 348))
    p2 = [pool(c2[(2 * p) * L:(2 * p + 1) * L],
               c2[(2 * p + 1) * L:(2 * p + 2) * L]) for p in range(2)]
    a2 = [mm(t, p2_ref[...]).astype(bf16) for t in p2]

    seq3 = a2 + [shift(a2[0]), shift(a2[1])]
    c3 = mm(conv_operand(seq3, 2), m3_ref[...].reshape(504, 144))
    rp = pool(c3[0:L], c3[L:2 * L])

    rg = mm(s6_ref[...], rp)
    hin = jnp.concatenate([rg[ip * B:(ip + 1) * B] for ip in range(6)],
                          axis=1).astype(bf16)
    hb = jnp.maximum(mm(hin, w1s_ref[...]), 0.0).astype(bf16)

    o_ref[...] = jnp.maximum(mm(hb, w2_ref[...]), 0.0)


@partial(jax.jit, static_argnames=("block_b",))
def _forward(x_nchw, m1a, m1b, p1a, p1b, m2, p2, m3, w1s, w2fc, block_b=128):
    n, cin, h, w = x_nchw.shape
    assert (cin, h, w) == (2, 64, 64)

    bb = max(1, min(block_b, n))
    n_pad = -(-n // bb) * bb
    L = 8 * bb

    if n_pad != n:
        x_nchw = jnp.pad(x_nchw, ((0, n_pad - n), (0, 0), (0, 0), (0, 0)))
    x6 = x_nchw.reshape(n_pad, 2, 8, 512)

    r1i = jnp.arange(384)[:, None]
    c1i = jnp.arange(384)[None, :]
    pm = c1i == ((r1i // 64) % 3) * 128 + 2 * (r1i % 64) + r1i // 192
    pm = jnp.where(pm, 0.5, 0.0).astype(jnp.bfloat16)

    row = jnp.arange(6 * bb)[:, None]
    col = jnp.arange(L)[None, :]
    s6 = (col == 8 * (row % bb) + row // bb).astype(jnp.bfloat16)

    grid = (n_pad // bb,)
    weights = [pm, m1a, m1b, p1a, p1b, m2, p2, m3, s6, w1s, w2fc]

    def const_spec(arr):
        nd = arr.ndim
        return pl.BlockSpec(arr.shape, lambda i, _nd=nd: (0,) * _nd)

    macs_per_img = (64 * 384 * 496 + 32 * 496 * 248 // 2
                    + 32 * 744 * 348 + 16 * 348 * 168
                    + 16 * 504 * 144 + 6 * 144 + 864 * 32 + 32 * 128)
    bytes_accessed = (x6.size * 4 + n_pad * 128 * 4
                      + sum(int(a.size) * 2 for a in weights))

    out = pl.pallas_call(
        _fused_kernel,
        out_shape=jax.ShapeDtypeStruct((n_pad, 128), jnp.float32),
        grid=grid,
        in_specs=[pl.BlockSpec((bb, 2, 8, 512), lambda i: (i, 0, 0, 0))]
                 + [const_spec(a) for a in weights],
        out_specs=pl.BlockSpec((bb, 128), lambda i: (i, 0)),
        compiler_params=pltpu.CompilerParams(
            dimension_semantics=("parallel",)),
        cost_estimate=pl.CostEstimate(
            flops=2 * macs_per_img * n_pad,
            transcendentals=0,
            bytes_accessed=int(bytes_accessed)),
    )(x6, *weights)

    return out[:n, :1]


def kernel(x, m1a, m1b, r1, p1a, p1b, m2, r2, p2, m3, r3, w1fc, w2fc):
    del r1, r2, r3
    half = jnp.asarray(0.5, jnp.bfloat16)
    return _forward(x, m1a, m1b, p1a, p1b, m2, p2 * half, m3,
                    (w1fc * half).reshape(864, 32), w2fc, block_b=128)


# speedup vs baseline: 1.0915x; 1.0010x over previous
from functools import partial

import jax
import jax.numpy as jnp
from jax.experimental import pallas as pl
from jax.experimental.pallas import tpu as pltpu


def _fused_kernel(x_ref, pm_ref, m1a_ref, m1b_ref, p1a_ref, p1b_ref,
                  m2_ref, p2_ref, m3_ref, s6_ref, w1s_ref, w2_ref, o_ref):
    f32, bf16 = jnp.float32, jnp.bfloat16
    B = x_ref.shape[0]
    L = 8 * B

    def mm(a, b):
        return jnp.dot(a, b, preferred_element_type=f32)

    def shift(a):
        return jnp.concatenate([a[1:], a[:1]], axis=0)

    def pool(u, v):
        a = jnp.maximum(u, 0.0).astype(bf16).astype(f32)
        b = jnp.maximum(v, 0.0).astype(bf16).astype(f32)
        return (a + b).astype(bf16)

    def conv_operand(seq, nphase):
        chunks = [jnp.concatenate(seq[k:k + 3], axis=1)
                  for k in range(nphase)]
        return jnp.concatenate(chunks, axis=0)

    mc1a = mm(pm_ref[...], m1a_ref[...].reshape(384, 256)).astype(bf16)
    mc1b = mm(pm_ref[...], m1b_ref[...].reshape(384, 240)).astype(bf16)

    ch = [x_ref[:, c].reshape(L, 512).astype(bf16) for c in range(2)]
    sh = [shift(t) for t in ch]
    cr = [jnp.concatenate([t[:, 64:], t[:, :64]], axis=1) for t in ch]
    def taps(k):
        if k % 2 == 0:
            return [ch[0][:, 64 * k:64 * k + 192],
                    ch[1][:, 64 * k:64 * k + 192]]
        return [cr[0][:, 64 * (k - 1):64 * (k - 1) + 192],
                cr[1][:, 64 * (k - 1):64 * (k - 1) + 192]]
    chunks = [jnp.concatenate(taps(k), axis=1) for k in range(6)]
    chunks.append(jnp.concatenate(
        [ch[0][:, 384:512], sh[0][:, 0:64],
         ch[1][:, 384:512], sh[1][:, 0:64]], axis=1))
    chunks.append(jnp.concatenate(
        [ch[0][:, 448:512], sh[0][:, 0:128],
         ch[1][:, 448:512], sh[1][:, 0:128]], axis=1))
    u1 = jnp.concatenate(chunks, axis=0)
    c1a = mm(u1, mc1a)
    c1b = mm(u1, mc1b)
    ta = jnp.concatenate([pool(c1a[(2 * p) * L:(2 * p + 1) * L],
                               c1a[(2 * p + 1) * L:(2 * p + 2) * L])
                          for p in range(4)], axis=0)
    tb = jnp.concatenate([pool(c1b[(2 * p) * L:(2 * p + 1) * L],
                               c1b[(2 * p + 1) * L:(2 * p + 2) * L])
                          for p in range(4)], axis=0)
    a1f = jnp.concatenate([mm(ta, p1a_ref[...]), mm(tb, p1b_ref[...])],
                          axis=1).astype(bf16)
    a1 = [a1f[p * L:(p + 1) * L] for p in range(4)]

    seq2 = a1 + [shift(a1[0]), shift(a1[1])]
    c2 = mm(conv_operand(seq2, 4), m2_ref[...].reshape(744, 348))
    p2 = [pool(c2[(2 * p) * L:(2 * p + 1) * L],
               c2[(2 * p + 1) * L:(2 * p + 2) * L]) for p in range(2)]
    a2 = [mm(t, p2_ref[...]).astype(bf16) for t in p2]

    seq3 = a2 + [shift(a2[0]), shift(a2[1])]
    c3 = mm(conv_operand(seq3, 2), m3_ref[...].reshape(504, 144))
    rp = pool(c3[0:L], c3[L:2 * L])

    rg = mm(s6_ref[...], rp)
    hin = jnp.concatenate([rg[ip * B:(ip + 1) * B] for ip in range(6)],
                          axis=1).astype(bf16)
    hb = jnp.maximum(mm(hin, w1s_ref[...]), 0.0).astype(bf16)

    o_ref[...] = jnp.maximum(mm(hb, w2_ref[...]), 0.0)


@partial(jax.jit, static_argnames=("block_b",))
def _forward(x_nchw, m1a, m1b, p1a, p1b, m2, p2, m3, w1s, w2fc, block_b=128):
    n, cin, h, w = x_nchw.shape
    assert (cin, h, w) == (2, 64, 64)

    bb = max(1, min(block_b, n))
    n_pad = -(-n // bb) * bb
    L = 8 * bb

    if n_pad != n:
        x_nchw = jnp.pad(x_nchw, ((0, n_pad - n), (0, 0), (0, 0), (0, 0)))
    x6 = x_nchw.reshape(n_pad, 2, 8, 512)

    r1i = jnp.arange(384)[:, None]
    c1i = jnp.arange(384)[None, :]
    pm = c1i == ((r1i // 64) % 3) * 128 + 2 * (r1i % 64) + r1i // 192
    pm = jnp.where(pm, 0.5, 0.0).astype(jnp.bfloat16)

    row = jnp.arange(6 * bb)[:, None]
    col = jnp.arange(L)[None, :]
    s6 = (col == 8 * (row % bb) + row // bb).astype(jnp.bfloat16)

    grid = (n_pad // bb,)
    weights = [pm, m1a, m1b, p1a, p1b, m2, p2, m3, s6, w1s, w2fc]

    def const_spec(arr):
        nd = arr.ndim
        return pl.BlockSpec(arr.shape, lambda i, _nd=nd: (0,) * _nd)

    macs_per_img = (64 * 384 * 496 + 32 * 496 * 248 // 2
                    + 32 * 744 * 348 + 16 * 348 * 168
                    + 16 * 504 * 144 + 6 * 144 + 864 * 32 + 32 * 128)
    bytes_accessed = (x6.size * 4 + n_pad * 128 * 4
                      + sum(int(a.size) * 2 for a in weights))

    out = pl.pallas_call(
        _fused_kernel,
        out_shape=jax.ShapeDtypeStruct((n_pad, 128), jnp.float32),
        grid=grid,
        in_specs=[pl.BlockSpec((bb, 2, 8, 512), lambda i: (i, 0, 0, 0))]
                 + [const_spec(a) for a in weights],
        out_specs=pl.BlockSpec((bb, 128), lambda i: (i, 0)),
        compiler_params=pltpu.CompilerParams(
            dimension_semantics=("parallel",)),
        cost_estimate=pl.CostEstimate(
            flops=2 * macs_per_img * n_pad,
            transcendentals=0,
            bytes_accessed=int(bytes_accessed)),
    )(x6, *weights)

    return out[:n, :1]


def kernel(x, m1a, m1b, r1, p1a, p1b, m2, r2, p2, m3, r3, w1fc, w2fc):
    del r1, r2, r3
    half = jnp.asarray(0.5, jnp.bfloat16)
    return _forward(x, m1a, m1b, p1a, p1b, m2, p2 * half, m3,
                    (w1fc * half).reshape(864, 32), w2fc, block_b=128)
